# baseline (speedup 1.0000x reference)
import functools

import jax
import jax.numpy as jnp
from jax import lax
from jax.experimental import pallas as pl
from jax.experimental.pallas import tpu as pltpu
from jax.experimental.pallas import tpu_sc as plsc

GB = 128


def _tc_body(idx_ref, img_ref, out_ref):
    g = pl.program_id(1)
    rel = idx_ref[0] - g * GB
    gmat = (jnp.broadcast_to(rel, (GB, GB))
            == lax.broadcasted_iota(jnp.int32, (GB, GB), 0)).astype(jnp.float32)
    x = img_ref[0]
    out_ref[0] = jax.lax.dot_general(
        gmat, x, (((1,), (0,)), ((), ())), preferred_element_type=jnp.float32)


def kernel(img, index_flat_inv):
    B, T, C = img.shape
    idx = index_flat_inv.astype(jnp.int32)
    ng = T // GB
    idx3 = idx.reshape(ng, 1, GB)

    return pl.pallas_call(
        _tc_body,
        grid=(B, ng),
        in_specs=[
            pl.BlockSpec((1, 1, GB), lambda b, g: (g, 0, 0)),
            pl.BlockSpec((1, GB, C), lambda b, g: (b, g, 0)),
        ],
        out_specs=pl.BlockSpec((1, GB, C), lambda b, g: (b, g, 0)),
        out_shape=jax.ShapeDtypeStruct((B, T, C), jnp.float32),
    )(idx3, img)


# speedup vs baseline: 2.0276x; 2.0276x over previous
import functools

import jax
import jax.numpy as jnp
from jax import lax
from jax.experimental import pallas as pl
from jax.experimental.pallas import tpu as pltpu
from jax.experimental.pallas import tpu_sc as plsc


def kernel(img, index_flat_inv):
    B, T, C = img.shape
    idx = index_flat_inv.astype(jnp.int32)

    info = plsc.get_sparse_core_info()
    NC, NS = info.num_cores, info.num_subcores
    NW = NC * NS
    assert B == NW

    CH = 64
    n_ch = T // CH
    idx2 = idx.reshape(n_ch, CH)

    mesh = plsc.VectorSubcoreMesh(core_axis_name="c", subcore_axis_name="s")

    @functools.partial(
        pl.kernel,
        mesh=mesh,
        out_type=jax.ShapeDtypeStruct((B, T, C), jnp.float32),
        scratch_types=[
            pltpu.VMEM((n_ch, CH), jnp.int32),
            pltpu.VMEM((CH, C), jnp.float32),
            pltpu.VMEM((CH, C), jnp.float32),
            pltpu.SemaphoreType.DMA,
            pltpu.SemaphoreType.DMA,
            pltpu.SemaphoreType.DMA,
            pltpu.SemaphoreType.DMA,
        ],
    )
    def k(img_hbm, idx_hbm, out_hbm, perm_v, rows0, rows1, rs0, rs1, ws0, ws1):
        wid = lax.axis_index("s") * NC + lax.axis_index("c")
        bufs, rsems, wsems = [rows0, rows1], [rs0, rs1], [ws0, ws1]
        pltpu.sync_copy(idx_hbm, perm_v)
        rd = [None, None]
        wr = [None, None]
        rd[0] = pltpu.async_copy(img_hbm.at[wid, pl.ds(0, CH)], bufs[0], rsems[0])
        for j in range(n_ch):
            cur, nxt = j % 2, (j + 1) % 2
            if j + 1 < n_ch:
                if wr[nxt] is not None:
                    wr[nxt].wait()
                rd[nxt] = pltpu.async_copy(
                    img_hbm.at[wid, pl.ds((j + 1) * CH, CH)], bufs[nxt], rsems[nxt])
            rd[cur].wait()
            wr[cur] = pltpu.async_copy(
                bufs[cur], out_hbm.at[wid].at[perm_v.at[j]], wsems[cur])
        for w in wr:
            if w is not None:
                w.wait()

    return k(img, idx2)


# speedup vs baseline: 2.0608x; 1.0164x over previous
import functools

import jax
import jax.numpy as jnp
from jax import lax
from jax.experimental import pallas as pl
from jax.experimental.pallas import tpu as pltpu
from jax.experimental.pallas import tpu_sc as plsc


def kernel(img, index_flat_inv):
    B, T, C = img.shape
    idx = index_flat_inv.astype(jnp.int32)

    info = plsc.get_sparse_core_info()
    NC, NS = info.num_cores, info.num_subcores
    NW = NC * NS
    assert B == NW

    CH = 32
    NBUF = 4
    n_ch = T // CH
    idx2 = idx.reshape(n_ch, CH)

    mesh = plsc.VectorSubcoreMesh(core_axis_name="c", subcore_axis_name="s")

    @functools.partial(
        pl.kernel,
        mesh=mesh,
        out_type=jax.ShapeDtypeStruct((B, T, C), jnp.float32),
        scratch_types=(
            [pltpu.VMEM((n_ch, CH), jnp.int32)]
            + [pltpu.VMEM((CH, C), jnp.float32)] * NBUF
            + [pltpu.SemaphoreType.DMA] * (2 * NBUF)
        ),
    )
    def k(img_hbm, idx_hbm, out_hbm, perm_v, *rest):
        bufs = rest[:NBUF]
        rsems = rest[NBUF:2 * NBUF]
        wsems = rest[2 * NBUF:]
        wid = lax.axis_index("s") * NC + lax.axis_index("c")
        rd = [None] * NBUF
        wr = [None] * NBUF
        for j in range(NBUF - 1):
            rd[j] = pltpu.async_copy(
                img_hbm.at[wid, pl.ds(j * CH, CH)], bufs[j], rsems[j])
        pltpu.sync_copy(idx_hbm, perm_v)
        for j in range(n_ch):
            cur = j % NBUF
            nj = j + NBUF - 1
            if nj < n_ch:
                b = nj % NBUF
                if wr[b] is not None:
                    wr[b].wait()
                    wr[b] = None
                rd[b] = pltpu.async_copy(
                    img_hbm.at[wid, pl.ds(nj * CH, CH)], bufs[b], rsems[b])
            rd[cur].wait()
            wr[cur] = pltpu.async_copy(
                bufs[cur], out_hbm.at[wid].at[perm_v.at[j]], wsems[cur])
        for w in wr:
            if w is not None:
                w.wait()

    return k(img, idx2)
